# Initial kernel scaffold; baseline (speedup 1.0000x reference)
import jax
import jax.numpy as jnp
from jax.experimental import pallas as pl


def kernel(pos, x, edge_index, edge_vec, W_sc1, W_lin1, W_fc1_1, W_fc1_2, W_lin2_s, W_lin2_v, W_sc2, W_l1s2, W_l1v2, W_fc2_1, W_fc2_2, W_lin2f):
    raise NotImplementedError("write your pallas kernel here")


# speedup vs baseline: 1.4872x; 1.4872x over previous
import functools

import jax
import jax.numpy as jnp
from jax import lax
from jax.experimental import pallas as pl
from jax.experimental.pallas import tpu as pltpu
from jax.experimental.pallas import tpu_sc as plsc

NN = 50000
NE = 1600000
T1 = 2000
TN = 2000
_RB = 10.0 ** 0.5 / 1.12
_INV_STEP = 9.0 / 2.5
_SQRT3 = 3.0 ** 0.5
_SQRT10 = 10.0 ** 0.5
_SQRT32 = 32.0 ** 0.5


def _sh_emb(ev):
    r = jnp.sqrt(jnp.sum(ev * ev, axis=1) + 1e-12)
    sh1 = _SQRT3 * (ev / r[:, None])
    centers = lax.broadcasted_iota(jnp.int32, (1, 10), 1).astype(jnp.float32)
    centers = centers * (2.5 / 9.0)
    diff = (r[:, None] - centers) * _INV_STEP
    emb = jnp.exp(-diff * diff) / 1.12 * _SQRT10
    return sh1, emb


def _prep_body(x_ref, wlin1_ref, out_ref):
    xl = jnp.dot(x_ref[...], wlin1_ref[...]) / 4.0
    out_ref[...] = jnp.concatenate(
        [xl, jnp.zeros((xl.shape[0], 112), jnp.float32)], axis=1)


def _prep(x, wlin1, n=NN, t=TN):
    return pl.pallas_call(
        _prep_body,
        grid=(n // t,),
        in_specs=[
            pl.BlockSpec((t, 16), lambda i: (i, 0)),
            pl.BlockSpec((16, 16), lambda i: (0, 0)),
        ],
        out_specs=pl.BlockSpec((t, 128), lambda i: (i, 0)),
        out_shape=jax.ShapeDtypeStruct((n, 128), jnp.float32),
    )(x, wlin1)


def _edge1_body(ev_ref, xs_ref, wh_ref, w12_ref, w22_ref, msg_ref, msgb_ref,
                w2_ref):
    ev = ev_ref[...]
    xs = xs_ref[...]
    sh1, emb = _sh_emb(ev)
    hh = jax.nn.silu(jnp.dot(emb, wh_ref[...]) / _SQRT10)
    h1 = hh[:, :100]
    h2 = hh[:, 100:]
    w = jnp.dot(h1, w12_ref[...]) / 10.0
    w2_ref[...] = jnp.dot(h2, w22_ref[...]) / 10.0
    ms = jnp.zeros((ev.shape[0], 32), jnp.float32)
    mv = jnp.zeros((ev.shape[0], 16), jnp.float32)
    for j in range(16):
        xj = xs[:, j:j + 1]
        ms = ms + xj * w[:, 32 * j:32 * j + 32]
        mv = mv + xj * w[:, 512 + 16 * j:512 + 16 * j + 16]
    ms = ms * 0.25
    mv = mv * 0.25
    v0 = mv * sh1[:, 0:1]
    v1 = mv * sh1[:, 1:2]
    v2 = mv * sh1[:, 2:3]
    pad = jnp.zeros((ev.shape[0], 16), jnp.float32)
    full = jnp.concatenate([ms, v0, v1, v2, pad], axis=1)
    msg_ref[0] = full[:, :24]
    msg_ref[1] = full[:, 24:48]
    msgb_ref[0] = full[:, 48:72]
    msgb_ref[1] = full[:, 72:96]


def _edge1(ev, xs, wh, w12, w22, e=NE, t=T1):
    return pl.pallas_call(
        _edge1_body,
        grid=(e // t,),
        in_specs=[
            pl.BlockSpec((t, 3), lambda i: (i, 0)),
            pl.BlockSpec((t, 16), lambda i: (i, 0)),
            pl.BlockSpec((10, 200), lambda i: (0, 0)),
            pl.BlockSpec((100, 768), lambda i: (0, 0)),
            pl.BlockSpec((100, 32), lambda i: (0, 0)),
        ],
        out_specs=[
            pl.BlockSpec((2, t, 24), lambda i: (0, i, 0)),
            pl.BlockSpec((2, t, 24), lambda i: (0, i, 0)),
            pl.BlockSpec((t, 32), lambda i: (i, 0)),
        ],
        out_shape=[
            jax.ShapeDtypeStruct((2, e, 24), jnp.float32),
            jax.ShapeDtypeStruct((2, e, 24), jnp.float32),
            jax.ShapeDtypeStruct((e, 32), jnp.float32),
        ],
    )(ev, xs, wh, w12, w22)


def _node_body(agg_ref, aggb_ref, x_ref, wsc1_ref, wl2s_ref, wl2v_ref,
               wl1s2_ref, wl1v2_ref, wsc2_ref, tab_ref, ssc_ref):
    full = jnp.concatenate([agg_ref[0], agg_ref[1],
                            aggb_ref[0], aggb_ref[1]], axis=1)
    sc = jnp.dot(x_ref[...], wsc1_ref[...]) / 4.0
    ags = full[:, :32] / _SQRT32
    s1 = sc + jnp.dot(ags, wl2s_ref[...]) / _SQRT32
    scal = jax.nn.silu(s1[:, :16])
    gates = jax.nn.sigmoid(s1[:, 16:32])
    s2 = jnp.dot(scal, wl1s2_ref[...]) / 4.0
    outs = [s2]
    for d in range(3):
        av_d = full[:, 32 + 16 * d:48 + 16 * d] / _SQRT32
        v1_d = jnp.dot(av_d, wl2v_ref[...]) / 4.0
        vec_d = v1_d * gates
        outs.append(jnp.dot(vec_d, wl1v2_ref[...]) / 4.0)
    outs.append(jnp.zeros((s2.shape[0], 64), jnp.float32))
    tab_ref[...] = jnp.concatenate(outs, axis=1)
    sc2 = jnp.dot(scal, wsc2_ref[...]) / 4.0

    @pl.when(pl.program_id(0) == 0)
    def _():
        ssc_ref[...] = jnp.zeros_like(ssc_ref)

    ssc_ref[...] += jnp.sum(sc2, axis=0, keepdims=True)


def _node(agg, aggb, x, wsc1, wl2s, wl2v, wl1s2, wl1v2, wsc2, n=NN, t=TN):
    return pl.pallas_call(
        _node_body,
        grid=(n // t,),
        in_specs=[
            pl.BlockSpec((2, t, 24), lambda i: (0, i, 0)),
            pl.BlockSpec((2, t, 24), lambda i: (0, i, 0)),
            pl.BlockSpec((t, 16), lambda i: (i, 0)),
            pl.BlockSpec((16, 32), lambda i: (0, 0)),
            pl.BlockSpec((32, 32), lambda i: (0, 0)),
            pl.BlockSpec((16, 16), lambda i: (0, 0)),
            pl.BlockSpec((16, 16), lambda i: (0, 0)),
            pl.BlockSpec((16, 16), lambda i: (0, 0)),
            pl.BlockSpec((16, 1), lambda i: (0, 0)),
        ],
        out_specs=[
            pl.BlockSpec((t, 128), lambda i: (i, 0)),
            pl.BlockSpec((1, 1), lambda i: (0, 0)),
        ],
        out_shape=[
            jax.ShapeDtypeStruct((n, 128), jnp.float32),
            jax.ShapeDtypeStruct((1, 1), jnp.float32),
        ],
    )(agg, aggb, x, wsc1, wl2s, wl2v, wl1s2, wl1v2, wsc2)


def _edge2_body(ev_ref, w2_ref, e2_ref, ssc_ref, wl2f_ref, out_ref, acc_ref):
    @pl.when(pl.program_id(0) == 0)
    def _():
        acc_ref[...] = jnp.zeros_like(acc_ref)

    ev = ev_ref[...]
    sh1, _ = _sh_emb(ev)
    w2 = w2_ref[...]
    e2 = e2_ref[...]
    mid0 = e2[:, :16] * w2[:, :16]
    q = (e2[:, 16:32] * sh1[:, 0:1] + e2[:, 32:48] * sh1[:, 1:2]
         + e2[:, 48:64] * sh1[:, 2:3])
    mid1 = q * w2[:, 16:32] / _SQRT3
    part = jnp.concatenate(
        [jnp.sum(mid0, axis=0, keepdims=True),
         jnp.sum(mid1, axis=0, keepdims=True)], axis=1)
    acc_ref[...] += part

    @pl.when(pl.program_id(0) == pl.num_programs(0) - 1)
    def _():
        summid = acc_ref[...] / _SQRT32
        out = ssc_ref[...] + jnp.dot(
            summid, wl2f_ref[...],
            precision=jax.lax.Precision.HIGHEST) / _SQRT32
        out_ref[...] = out / (50000.0 ** 0.5)


def _edge2(ev, w2, e2, ssc, wl2f, e=NE, t=T1):
    return pl.pallas_call(
        _edge2_body,
        grid=(e // t,),
        in_specs=[
            pl.BlockSpec((t, 3), lambda i: (i, 0)),
            pl.BlockSpec((t, 32), lambda i: (i, 0)),
            pl.BlockSpec((t, 64), lambda i: (i, 0)),
            pl.BlockSpec((1, 1), lambda i: (0, 0)),
            pl.BlockSpec((32, 1), lambda i: (0, 0)),
        ],
        out_specs=pl.BlockSpec((1, 1), lambda i: (0, 0)),
        out_shape=jax.ShapeDtypeStruct((1, 1), jnp.float32),
        scratch_shapes=[pltpu.VMEM((1, 32), jnp.float32)],
    )(ev, w2, e2, ssc, wl2f)


def _make_gather(d, n=NN, e=NE):
    mesh = plsc.VectorSubcoreMesh(core_axis_name="c", subcore_axis_name="s")
    w = 128
    nwin = e // w
    base_full = nwin // 32
    n_extra = nwin - base_full * 32

    @functools.partial(
        pl.kernel,
        out_type=jax.ShapeDtypeStruct((e, d), jnp.float32),
        mesh=mesh,
        scratch_types=[
            pltpu.VMEM((w,), jnp.int32),
            pltpu.VMEM((w, 128), jnp.float32),
            pltpu.VMEM((w, d), jnp.float32),
            pltpu.SemaphoreType.DMA,
        ],
    )
    def k(tbl, ei, out, idx_v, rows_v, small_v, sem):
        wid = lax.axis_index("s") * 2 + lax.axis_index("c")

        def win(widx):
            b = (wid + 32 * widx) * w
            pltpu.sync_copy(ei.at[pl.ds(b, w)], idx_v)
            pltpu.async_copy(tbl.at[idx_v], rows_v, sem).wait()
            for i in range(w):
                for m in range(d // 16):
                    small_v[i, pl.ds(16 * m, 16)] = rows_v[i, pl.ds(16 * m, 16)]
            pltpu.sync_copy(small_v, out.at[pl.ds(b, w)])

        def body(i, _):
            win(i)
            return 0

        lax.fori_loop(0, base_full, body, 0, unroll=False)

        @pl.when(wid < n_extra)
        def _():
            win(base_full)

    return k


_make_gather = functools.lru_cache(maxsize=None)(_make_gather)


def kernel(pos, x, edge_index, edge_vec, W_sc1, W_lin1, W_fc1_1, W_fc1_2,
           W_lin2_s, W_lin2_v, W_sc2, W_l1s2, W_l1v2, W_fc2_1, W_fc2_2,
           W_lin2f):
    wh = jnp.concatenate([W_fc1_1, W_fc2_1], axis=1)
    src = edge_index[0]
    dst = edge_index[1]
    xl128 = _prep(x, W_lin1)
    xs = _make_gather(16)(xl128, src)
    msga, msgb, w2 = _edge1(edge_vec, xs, wh, W_fc1_2, W_fc2_2)
    fa = jnp.concatenate([msga[0], msga[1]], axis=1)
    fb = jnp.concatenate([msgb[0], msgb[1]], axis=1)
    sa = jax.ops.segment_sum(fa, dst, num_segments=NN)
    sb = jax.ops.segment_sum(fb, dst, num_segments=NN)
    agg = jnp.stack([sa[:, :24], sa[:, 24:]], axis=0)
    aggb = jnp.stack([sb[:, :24], sb[:, 24:]], axis=0)
    tab2, ssc = _node(agg, aggb, x, W_sc1, W_lin2_s, W_lin2_v, W_l1s2,
                      W_l1v2, W_sc2)
    e2 = _make_gather(64)(tab2, src)
    out = _edge2(edge_vec, w2, e2, ssc, W_lin2f)
    return out


# speedup vs baseline: 1.9139x; 1.2869x over previous
import functools

import jax
import jax.numpy as jnp
from jax import lax
from jax.experimental import pallas as pl
from jax.experimental.pallas import tpu as pltpu
from jax.experimental.pallas import tpu_sc as plsc

NN = 50000
NE = 1600000
T1 = 2000
TN = 2000
_RB = 10.0 ** 0.5 / 1.12
_INV_STEP = 9.0 / 2.5
_SQRT3 = 3.0 ** 0.5
_SQRT10 = 10.0 ** 0.5
_SQRT32 = 32.0 ** 0.5


def _sh_emb(ev):
    r = jnp.sqrt(jnp.sum(ev * ev, axis=1) + 1e-12)
    sh1 = _SQRT3 * (ev / r[:, None])
    centers = lax.broadcasted_iota(jnp.int32, (1, 10), 1).astype(jnp.float32)
    centers = centers * (2.5 / 9.0)
    diff = (r[:, None] - centers) * _INV_STEP
    emb = jnp.exp(-diff * diff) / 1.12 * _SQRT10
    return sh1, emb


def _prep_body(x_ref, wlin1_ref, out_ref):
    xl = jnp.dot(x_ref[...], wlin1_ref[...]) / 4.0
    out_ref[...] = jnp.concatenate(
        [xl, jnp.zeros((xl.shape[0], 112), jnp.float32)], axis=1)


def _prep(x, wlin1, n=NN, t=TN):
    return pl.pallas_call(
        _prep_body,
        grid=(n // t,),
        in_specs=[
            pl.BlockSpec((t, 16), lambda i: (i, 0)),
            pl.BlockSpec((16, 16), lambda i: (0, 0)),
        ],
        out_specs=pl.BlockSpec((t, 128), lambda i: (i, 0)),
        out_shape=jax.ShapeDtypeStruct((n, 128), jnp.float32),
    )(x, wlin1)


def _edge1_body(ev_ref, xs_ref, wh_ref, w12_ref, w22_ref, msg_ref, w2_ref):
    ev = ev_ref[...]
    xs = xs_ref[...]
    sh1, emb = _sh_emb(ev)
    hh = jax.nn.silu(jnp.dot(emb, wh_ref[...]) / _SQRT10)
    h1 = hh[:, :100]
    h2 = hh[:, 100:]
    w = jnp.dot(h1, w12_ref[...]) / 10.0
    w2_ref[...] = jnp.dot(h2, w22_ref[...]) / 10.0
    ms = jnp.zeros((ev.shape[0], 32), jnp.float32)
    mv = jnp.zeros((ev.shape[0], 16), jnp.float32)
    for j in range(16):
        xj = xs[:, j:j + 1]
        ms = ms + xj * w[:, 32 * j:32 * j + 32]
        mv = mv + xj * w[:, 512 + 16 * j:512 + 16 * j + 16]
    ms = ms * 0.25
    mv = mv * 0.25
    v0 = mv * sh1[:, 0:1]
    v1 = mv * sh1[:, 1:2]
    v2 = mv * sh1[:, 2:3]
    pad = jnp.zeros((ev.shape[0], 16), jnp.float32)
    msg_ref[...] = jnp.concatenate([ms, v0, v1, v2, pad], axis=1)


def _edge1(ev, xs, wh, w12, w22, e=NE, t=T1):
    return pl.pallas_call(
        _edge1_body,
        grid=(e // t,),
        in_specs=[
            pl.BlockSpec((t, 3), lambda i: (i, 0)),
            pl.BlockSpec((t, 16), lambda i: (i, 0)),
            pl.BlockSpec((10, 200), lambda i: (0, 0)),
            pl.BlockSpec((100, 768), lambda i: (0, 0)),
            pl.BlockSpec((100, 32), lambda i: (0, 0)),
        ],
        out_specs=[
            pl.BlockSpec((t, 96), lambda i: (i, 0)),
            pl.BlockSpec((t, 32), lambda i: (i, 0)),
        ],
        out_shape=[
            jax.ShapeDtypeStruct((e, 96), jnp.float32),
            jax.ShapeDtypeStruct((e, 32), jnp.float32),
        ],
    )(ev, xs, wh, w12, w22)


def _node_body(agg_ref, x_ref, wsc1_ref, wl2s_ref, wl2v_ref,
               wl1s2_ref, wl1v2_ref, wsc2_ref, tab_ref, ssc_ref):
    full = agg_ref[...]
    sc = jnp.dot(x_ref[...], wsc1_ref[...]) / 4.0
    ags = full[:, :32] / _SQRT32
    s1 = sc + jnp.dot(ags, wl2s_ref[...]) / _SQRT32
    scal = jax.nn.silu(s1[:, :16])
    gates = jax.nn.sigmoid(s1[:, 16:32])
    s2 = jnp.dot(scal, wl1s2_ref[...]) / 4.0
    outs = [s2]
    for d in range(3):
        av_d = full[:, 32 + 16 * d:48 + 16 * d] / _SQRT32
        v1_d = jnp.dot(av_d, wl2v_ref[...]) / 4.0
        vec_d = v1_d * gates
        outs.append(jnp.dot(vec_d, wl1v2_ref[...]) / 4.0)
    outs.append(jnp.zeros((s2.shape[0], 64), jnp.float32))
    tab_ref[...] = jnp.concatenate(outs, axis=1)
    sc2 = jnp.dot(scal, wsc2_ref[...]) / 4.0

    @pl.when(pl.program_id(0) == 0)
    def _():
        ssc_ref[...] = jnp.zeros_like(ssc_ref)

    ssc_ref[...] += jnp.sum(sc2, axis=0, keepdims=True)


def _node(agg, x, wsc1, wl2s, wl2v, wl1s2, wl1v2, wsc2, n=NN, t=TN):
    return pl.pallas_call(
        _node_body,
        grid=(n // t,),
        in_specs=[
            pl.BlockSpec((t, 96), lambda i: (i, 0)),
            pl.BlockSpec((t, 16), lambda i: (i, 0)),
            pl.BlockSpec((16, 32), lambda i: (0, 0)),
            pl.BlockSpec((32, 32), lambda i: (0, 0)),
            pl.BlockSpec((16, 16), lambda i: (0, 0)),
            pl.BlockSpec((16, 16), lambda i: (0, 0)),
            pl.BlockSpec((16, 16), lambda i: (0, 0)),
            pl.BlockSpec((16, 1), lambda i: (0, 0)),
        ],
        out_specs=[
            pl.BlockSpec((t, 128), lambda i: (i, 0)),
            pl.BlockSpec((1, 1), lambda i: (0, 0)),
        ],
        out_shape=[
            jax.ShapeDtypeStruct((n, 128), jnp.float32),
            jax.ShapeDtypeStruct((1, 1), jnp.float32),
        ],
    )(agg, x, wsc1, wl2s, wl2v, wl1s2, wl1v2, wsc2)


def _edge2_body(ev_ref, w2_ref, e2_ref, ssc_ref, wl2f_ref, out_ref, acc_ref):
    @pl.when(pl.program_id(0) == 0)
    def _():
        acc_ref[...] = jnp.zeros_like(acc_ref)

    ev = ev_ref[...]
    sh1, _ = _sh_emb(ev)
    w2 = w2_ref[...]
    e2 = e2_ref[...]
    mid0 = e2[:, :16] * w2[:, :16]
    q = (e2[:, 16:32] * sh1[:, 0:1] + e2[:, 32:48] * sh1[:, 1:2]
         + e2[:, 48:64] * sh1[:, 2:3])
    mid1 = q * w2[:, 16:32] / _SQRT3
    part = jnp.concatenate(
        [jnp.sum(mid0, axis=0, keepdims=True),
         jnp.sum(mid1, axis=0, keepdims=True)], axis=1)
    acc_ref[...] += part

    @pl.when(pl.program_id(0) == pl.num_programs(0) - 1)
    def _():
        summid = acc_ref[...] / _SQRT32
        out = ssc_ref[...] + jnp.dot(
            summid, wl2f_ref[...],
            precision=jax.lax.Precision.HIGHEST) / _SQRT32
        out_ref[...] = out / (50000.0 ** 0.5)


def _edge2(ev, w2, e2, ssc, wl2f, e=NE, t=T1):
    return pl.pallas_call(
        _edge2_body,
        grid=(e // t,),
        in_specs=[
            pl.BlockSpec((t, 3), lambda i: (i, 0)),
            pl.BlockSpec((t, 32), lambda i: (i, 0)),
            pl.BlockSpec((t, 64), lambda i: (i, 0)),
            pl.BlockSpec((1, 1), lambda i: (0, 0)),
            pl.BlockSpec((32, 1), lambda i: (0, 0)),
        ],
        out_specs=pl.BlockSpec((1, 1), lambda i: (0, 0)),
        out_shape=jax.ShapeDtypeStruct((1, 1), jnp.float32),
        scratch_shapes=[pltpu.VMEM((1, 32), jnp.float32)],
    )(ev, w2, e2, ssc, wl2f)


def _make_gather(d, n=NN, e=NE):
    mesh = plsc.VectorSubcoreMesh(core_axis_name="c", subcore_axis_name="s")
    w = 128
    nwin = e // w
    base_full = nwin // 32
    n_extra = nwin - base_full * 32

    @functools.partial(
        pl.kernel,
        out_type=jax.ShapeDtypeStruct((e, d), jnp.float32),
        mesh=mesh,
        scratch_types=[
            pltpu.VMEM((w,), jnp.int32),
            pltpu.VMEM((w, 128), jnp.float32),
            pltpu.VMEM((w, d), jnp.float32),
            pltpu.SemaphoreType.DMA,
        ],
    )
    def k(tbl, ei, out, idx_v, rows_v, small_v, sem):
        wid = lax.axis_index("s") * 2 + lax.axis_index("c")

        def win(widx):
            b = (wid + 32 * widx) * w
            pltpu.sync_copy(ei.at[pl.ds(b, w)], idx_v)
            pltpu.async_copy(tbl.at[idx_v], rows_v, sem).wait()
            for i in range(w):
                for m in range(d // 16):
                    small_v[i, pl.ds(16 * m, 16)] = rows_v[i, pl.ds(16 * m, 16)]
            pltpu.sync_copy(small_v, out.at[pl.ds(b, w)])

        def body(i, _):
            win(i)
            return 0

        lax.fori_loop(0, base_full, body, 0, unroll=False)

        @pl.when(wid < n_extra)
        def _():
            win(base_full)

    return k


_make_gather = functools.lru_cache(maxsize=None)(_make_gather)


def kernel(pos, x, edge_index, edge_vec, W_sc1, W_lin1, W_fc1_1, W_fc1_2,
           W_lin2_s, W_lin2_v, W_sc2, W_l1s2, W_l1v2, W_fc2_1, W_fc2_2,
           W_lin2f):
    wh = jnp.concatenate([W_fc1_1, W_fc2_1], axis=1)
    src = edge_index[0]
    dst = edge_index[1]
    xl128 = _prep(x, W_lin1)
    xs = _make_gather(16)(xl128, src)
    msg, w2 = _edge1(edge_vec, xs, wh, W_fc1_2, W_fc2_2)
    agg = jax.ops.segment_sum(msg, dst, num_segments=NN)
    tab2, ssc = _node(agg, x, W_sc1, W_lin2_s, W_lin2_v, W_l1s2,
                      W_l1v2, W_sc2)
    e2 = _make_gather(64)(tab2, src)
    out = _edge2(edge_vec, w2, e2, ssc, W_lin2f)
    return out
